# baseline (speedup 1.0000x reference)
import functools

import jax
import jax.numpy as jnp
from jax import lax
from jax.experimental import pallas as pl
from jax.experimental.pallas import tpu as pltpu
from jax.experimental.pallas import tpu_sc as plsc

D = 64
DP = 128
T = 4096
B = 4
V = 256
ROWS = B * T
NC, NS, L = 2, 16, 16
NW = NC * NS
RPW = ROWS // NW

BLK = 1024


_MESH = plsc.VectorSubcoreMesh(core_axis_name="c", subcore_axis_name="s")


@functools.partial(
    pl.kernel,
    out_type=jax.ShapeDtypeStruct((ROWS, DP), jnp.float32),
    mesh=_MESH,
    scratch_types=[
        pltpu.VMEM((RPW,), jnp.int32),
        pltpu.VMEM((RPW, DP), jnp.float32),
        pltpu.SemaphoreType.DMA,
    ],
)
def _sc_gather(x_hbm, table_hbm, out_hbm, idx_v, rows_v, sem):
    wid = lax.axis_index("s") * NC + lax.axis_index("c")
    base = wid * RPW
    pltpu.sync_copy(x_hbm.at[pl.ds(base, RPW)], idx_v)
    pltpu.async_copy(table_hbm.at[idx_v], rows_v, sem).wait()
    pltpu.sync_copy(rows_v, out_hbm.at[pl.ds(base, RPW)])


def _finish_body(g_ref, pos_ref, w_ref, b_ref, y_ref):
    h = g_ref[:, :D] + pos_ref[...]
    dn = (((1,), (1,)), ((), ()))
    y_ref[...] = lax.dot_general(h, w_ref[...], dn,
                                 preferred_element_type=jnp.float32) + b_ref[...]


def _tc_finish(g, pos_embed, W, b2d):
    nblk = ROWS // BLK
    return pl.pallas_call(
        _finish_body,
        grid=(nblk,),
        in_specs=[
            pl.BlockSpec((BLK, DP), lambda i: (i, 0)),
            pl.BlockSpec((BLK, D), lambda i: (i % (T // BLK), 0)),
            pl.BlockSpec((D, D), lambda i: (0, 0)),
            pl.BlockSpec((1, D), lambda i: (0, 0)),
        ],
        out_specs=pl.BlockSpec((BLK, D), lambda i: (i, 0)),
        out_shape=jax.ShapeDtypeStruct((ROWS, D), jnp.float32),
    )(g, pos_embed, W, b2d)


def kernel(x, byte_embed, pos_embed, W, b):
    x_flat = x.reshape(ROWS).astype(jnp.int32)
    table = jnp.pad(byte_embed, ((0, 0), (0, DP - D)))
    g = _sc_gather(x_flat, table)
    y = _tc_finish(g, pos_embed, W, b.reshape(1, D))
    return y.reshape(B, T, D)


# speedup vs baseline: 1.0053x; 1.0053x over previous
import functools

import jax
import jax.numpy as jnp
from jax import lax
from jax.experimental import pallas as pl
from jax.experimental.pallas import tpu as pltpu
from jax.experimental.pallas import tpu_sc as plsc

D = 64
DP = 128
T = 4096
B = 4
V = 256
ROWS = B * T
NC, NS, L = 2, 16, 16
NW = NC * NS
RPW = ROWS // NW

BLK = 1024


_MESH = plsc.VectorSubcoreMesh(core_axis_name="c", subcore_axis_name="s")


@functools.partial(
    pl.kernel,
    out_type=jax.ShapeDtypeStruct((ROWS, DP), jnp.float32),
    mesh=_MESH,
    scratch_types=[
        pltpu.VMEM((RPW,), jnp.int32),
        pltpu.VMEM((RPW, DP), jnp.float32),
        pltpu.SemaphoreType.DMA,
    ],
)
def _sc_gather(x_hbm, table_hbm, out_hbm, idx_v, rows_v, sem):
    wid = lax.axis_index("s") * NC + lax.axis_index("c")
    base = wid * RPW
    pltpu.sync_copy(x_hbm.at[pl.ds(base, RPW)], idx_v)
    pltpu.async_copy(table_hbm.at[idx_v], rows_v, sem).wait()
    pltpu.sync_copy(rows_v, out_hbm.at[pl.ds(base, RPW)])


def _finish_body(g_ref, pos_ref, w_ref, b_ref, y_ref):
    h = g_ref[:, :D] + pos_ref[...]
    dn = (((1,), (1,)), ((), ()))
    y_ref[0] = lax.dot_general(h, w_ref[...], dn,
                               preferred_element_type=jnp.float32) + b_ref[...]


def _tc_finish(g, pos_embed, W, b2d):
    nblk_t = T // BLK
    return pl.pallas_call(
        _finish_body,
        grid=(B, nblk_t),
        in_specs=[
            pl.BlockSpec((BLK, DP), lambda b, i: (b * nblk_t + i, 0)),
            pl.BlockSpec((BLK, D), lambda b, i: (i, 0)),
            pl.BlockSpec((D, D), lambda b, i: (0, 0)),
            pl.BlockSpec((1, D), lambda b, i: (0, 0)),
        ],
        out_specs=pl.BlockSpec((1, BLK, D), lambda b, i: (b, i, 0)),
        out_shape=jax.ShapeDtypeStruct((B, T, D), jnp.float32),
    )(g, pos_embed, W, b2d)


def kernel(x, byte_embed, pos_embed, W, b):
    x_flat = x.reshape(ROWS).astype(jnp.int32)
    table = jnp.pad(byte_embed, ((0, 0), (0, DP - D)))
    g = _sc_gather(x_flat, table)
    return _tc_finish(g, pos_embed, W, b.reshape(1, D))


# speedup vs baseline: 1.0201x; 1.0148x over previous
import functools

import jax
import jax.numpy as jnp
from jax import lax
from jax.experimental import pallas as pl
from jax.experimental.pallas import tpu as pltpu
from jax.experimental.pallas import tpu_sc as plsc

D = 64
DP = 128
T = 4096
B = 4
V = 256
ROWS = B * T
NC, NS, L = 2, 16, 16
NW = NC * NS
RPW = ROWS // NW

BLK = 1024


_MESH = plsc.VectorSubcoreMesh(core_axis_name="c", subcore_axis_name="s")


@functools.partial(
    pl.kernel,
    out_type=jax.ShapeDtypeStruct((ROWS, DP), jnp.float32),
    mesh=_MESH,
    scratch_types=[
        pltpu.VMEM((RPW,), jnp.int32),
        pltpu.VMEM((RPW, DP), jnp.float32),
        pltpu.SemaphoreType.DMA,
    ],
)
def _sc_gather(x_hbm, table_hbm, out_hbm, idx_v, rows_v, sem):
    wid = lax.axis_index("s") * NC + lax.axis_index("c")
    base = wid * RPW
    pltpu.sync_copy(x_hbm.at[pl.ds(base, RPW)], idx_v)
    pltpu.async_copy(table_hbm.at[idx_v], rows_v, sem).wait()
    pltpu.sync_copy(rows_v, out_hbm.at[pl.ds(base, RPW)])


def _finish_body(g_ref, pos_ref, w_ref, b_ref, y_ref):
    h = g_ref[:, :D] + pos_ref[...]
    dn = (((1,), (1,)), ((), ()))
    y_ref[0] = lax.dot_general(h, w_ref[...], dn,
                               preferred_element_type=jnp.float32) + b_ref[...]


def _tc_finish(g, pos_embed, W, b2d):
    nblk_t = T // BLK
    return pl.pallas_call(
        _finish_body,
        grid=(nblk_t, B),
        in_specs=[
            pl.BlockSpec((BLK, DP), lambda i, b: (b * nblk_t + i, 0)),
            pl.BlockSpec((BLK, D), lambda i, b: (i, 0)),
            pl.BlockSpec((D, D), lambda i, b: (0, 0)),
            pl.BlockSpec((1, D), lambda i, b: (0, 0)),
        ],
        out_specs=pl.BlockSpec((1, BLK, D), lambda i, b: (b, i, 0)),
        out_shape=jax.ShapeDtypeStruct((B, T, D), jnp.float32),
    )(g, pos_embed, W, b2d)


def kernel(x, byte_embed, pos_embed, W, b):
    x_flat = x.reshape(ROWS).astype(jnp.int32)
    table = jnp.pad(byte_embed, ((0, 0), (0, DP - D)))
    g = _sc_gather(x_flat, table)
    return _tc_finish(g, pos_embed, W, b.reshape(1, D))


# speedup vs baseline: 1.1011x; 1.0793x over previous
import functools

import jax
import jax.numpy as jnp
from jax import lax
from jax.experimental import pallas as pl
from jax.experimental.pallas import tpu as pltpu
from jax.experimental.pallas import tpu_sc as plsc

D = 64
DP = 128
T = 4096
B = 4
V = 256
ROWS = B * T
NC, NS, L = 2, 16, 16
NW = NC * NS
RPW = ROWS // NW

BLK = 2048


_MESH = plsc.VectorSubcoreMesh(core_axis_name="c", subcore_axis_name="s")


@functools.partial(
    pl.kernel,
    out_type=jax.ShapeDtypeStruct((ROWS, DP), jnp.float32),
    mesh=_MESH,
    scratch_types=[
        pltpu.VMEM((RPW,), jnp.int32),
        pltpu.VMEM((RPW, DP), jnp.float32),
        pltpu.SemaphoreType.DMA,
    ],
)
def _sc_gather(x_hbm, table_hbm, out_hbm, idx_v, rows_v, sem):
    wid = lax.axis_index("s") * NC + lax.axis_index("c")
    base = wid * RPW
    pltpu.sync_copy(x_hbm.at[pl.ds(base, RPW)], idx_v)
    pltpu.async_copy(table_hbm.at[idx_v], rows_v, sem).wait()
    pltpu.sync_copy(rows_v, out_hbm.at[pl.ds(base, RPW)])


def _finish_body(g_ref, pos_ref, w_ref, b_ref, y_ref):
    h = g_ref[:, :D] + pos_ref[...]
    dn = (((1,), (1,)), ((), ()))
    y_ref[0] = lax.dot_general(h, w_ref[...], dn,
                               preferred_element_type=jnp.float32) + b_ref[...]


def _tc_finish(g, pos_embed, W, b2d):
    nblk_t = T // BLK
    return pl.pallas_call(
        _finish_body,
        grid=(nblk_t, B),
        in_specs=[
            pl.BlockSpec((BLK, DP), lambda i, b: (b * nblk_t + i, 0)),
            pl.BlockSpec((BLK, D), lambda i, b: (i, 0)),
            pl.BlockSpec((D, D), lambda i, b: (0, 0)),
            pl.BlockSpec((1, D), lambda i, b: (0, 0)),
        ],
        out_specs=pl.BlockSpec((1, BLK, D), lambda i, b: (b, i, 0)),
        out_shape=jax.ShapeDtypeStruct((B, T, D), jnp.float32),
    )(g, pos_embed, W, b2d)


def kernel(x, byte_embed, pos_embed, W, b):
    x_flat = x.reshape(ROWS).astype(jnp.int32)
    table = jnp.pad(byte_embed, ((0, 0), (0, DP - D)))
    g = _sc_gather(x_flat, table)
    return _tc_finish(g, pos_embed, W, b.reshape(1, D))


# speedup vs baseline: 1.1614x; 1.0548x over previous
import functools

import jax
import jax.numpy as jnp
from jax import lax
from jax.experimental import pallas as pl
from jax.experimental.pallas import tpu as pltpu
from jax.experimental.pallas import tpu_sc as plsc

D = 64
DP = 128
T = 4096
B = 4
V = 256
ROWS = B * T
NC, NS, L = 2, 16, 16
NW = NC * NS
RPW = ROWS // NW

BLK = 4096


_MESH = plsc.VectorSubcoreMesh(core_axis_name="c", subcore_axis_name="s")


@functools.partial(
    pl.kernel,
    out_type=jax.ShapeDtypeStruct((ROWS, DP), jnp.float32),
    mesh=_MESH,
    scratch_types=[
        pltpu.VMEM((RPW,), jnp.int32),
        pltpu.VMEM((RPW, DP), jnp.float32),
        pltpu.SemaphoreType.DMA,
    ],
)
def _sc_gather(x_hbm, table_hbm, out_hbm, idx_v, rows_v, sem):
    wid = lax.axis_index("s") * NC + lax.axis_index("c")
    base = wid * RPW
    pltpu.sync_copy(x_hbm.at[pl.ds(base, RPW)], idx_v)
    pltpu.async_copy(table_hbm.at[idx_v], rows_v, sem).wait()
    pltpu.sync_copy(rows_v, out_hbm.at[pl.ds(base, RPW)])


def _finish_body(g_ref, pos_ref, w_ref, b_ref, y_ref):
    h = g_ref[:, :D] + pos_ref[...]
    dn = (((1,), (1,)), ((), ()))
    y_ref[0] = lax.dot_general(h, w_ref[...], dn,
                               preferred_element_type=jnp.float32) + b_ref[...]


def _tc_finish(g, pos_embed, W, b2d):
    nblk_t = T // BLK
    return pl.pallas_call(
        _finish_body,
        grid=(nblk_t, B),
        in_specs=[
            pl.BlockSpec((BLK, DP), lambda i, b: (b * nblk_t + i, 0)),
            pl.BlockSpec((BLK, D), lambda i, b: (i, 0)),
            pl.BlockSpec((D, D), lambda i, b: (0, 0)),
            pl.BlockSpec((1, D), lambda i, b: (0, 0)),
        ],
        out_specs=pl.BlockSpec((1, BLK, D), lambda i, b: (b, i, 0)),
        out_shape=jax.ShapeDtypeStruct((B, T, D), jnp.float32),
    )(g, pos_embed, W, b2d)


def kernel(x, byte_embed, pos_embed, W, b):
    x_flat = x.reshape(ROWS).astype(jnp.int32)
    table = jnp.pad(byte_embed, ((0, 0), (0, DP - D)))
    g = _sc_gather(x_flat, table)
    return _tc_finish(g, pos_embed, W, b.reshape(1, D))
